# baseline (speedup 1.0000x reference)
import functools

import jax
import jax.numpy as jnp
from jax import lax
from jax.experimental import pallas as pl
from jax.experimental.pallas import tpu as pltpu
from jax.experimental.pallas import tpu_sc as plsc

IGNORE_INDEX = -100
LANES = 16
IDX_MINOR = 128


def _sc_workers():
    try:
        info = plsc.get_sparse_core_info()
        return info.num_cores, info.num_subcores
    except Exception:
        return 2, 16


@functools.lru_cache(maxsize=None)
def _build(B: int, C: int):
    NC, NS = _sc_workers()
    NW = NC * NS
    assert B % (NW * IDX_MINOR) == 0 and C % 8 == 0
    assert B & (B - 1) == 0
    SH_T = (8 * B).bit_length() - 1
    b_per_w = B // NW
    n_str = b_per_w // IDX_MINOR
    mesh = plsc.VectorSubcoreMesh(core_axis_name="c", subcore_axis_name="s",
                                  num_cores=NC)

    @functools.partial(
        pl.kernel,
        out_type=jax.ShapeDtypeStruct((B,), jnp.float32),
        mesh=mesh,
        scratch_types=[
            pltpu.VMEM((b_per_w,), jnp.int32),
            pltpu.VMEM((b_per_w,), jnp.int32),
            pltpu.VMEM((b_per_w,), jnp.int32),
            pltpu.VMEM((b_per_w,), jnp.float32),
            pltpu.VMEM((b_per_w,), jnp.float32),
            pltpu.VMEM((b_per_w,), jnp.float32),
            pltpu.SemaphoreType.DMA,
        ],
    )
    def nll_kernel(xf_hbm, tgt_hbm, w_hbm, out_hbm,
                   tgt_v, idx_v, safe_v, picked_v, wpick_v, out_v, sem):
        wid = lax.axis_index("s") * NC + lax.axis_index("c")
        base = wid * b_per_w

        pltpu.sync_copy(tgt_hbm.at[pl.ds(base, b_per_w)], tgt_v)

        lane = lax.iota(jnp.int32, LANES)

        def idx_body(j, _):
            o = j * LANES
            tv = tgt_v[pl.ds(o, LANES)]
            safe_t = jnp.where(tv == IGNORE_INDEX, 0, tv)
            rows = base + o + lane
            safe_v[pl.ds(o, LANES)] = safe_t
            idx_v[pl.ds(o, LANES)] = (
                ((safe_t >> 3) << SH_T)
                + ((rows >> 7) << 10)
                + ((safe_t & 7) << 7)
                + (rows & 127))
            return _

        copies = []
        for r in range(n_str):
            lo = r * (IDX_MINOR // LANES)
            lax.fori_loop(lo, lo + IDX_MINOR // LANES, idx_body, None)
            s = pl.ds(r * IDX_MINOR, IDX_MINOR)
            copies.append(
                pltpu.async_copy(xf_hbm.at[idx_v.at[s]], picked_v.at[s], sem))
            copies.append(
                pltpu.async_copy(w_hbm.at[safe_v.at[s]], wpick_v.at[s], sem))

        def out_body(j, _):
            o = j * LANES
            tv = tgt_v[pl.ds(o, LANES)]
            valid = tv != IGNORE_INDEX
            wv = wpick_v[pl.ds(o, LANES)]
            pk = picked_v[pl.ds(o, LANES)]
            out_v[pl.ds(o, LANES)] = jnp.where(
                valid, -(wv * pk), jnp.float32(0.0))
            return _

        for r in range(n_str):
            copies[2 * r].wait()
            copies[2 * r + 1].wait()
            lo = r * (IDX_MINOR // LANES)
            lax.fori_loop(lo, lo + IDX_MINOR // LANES, out_body, None)
            s = pl.ds(r * IDX_MINOR, IDX_MINOR)
            pltpu.sync_copy(out_v.at[s], out_hbm.at[pl.ds(base + r * IDX_MINOR,
                                                          IDX_MINOR)])

    return nll_kernel


def kernel(x, target, weight):
    B, C = x.shape
    fn = _build(B, C)
    x_flat = (x.T.reshape(C // 8, 8, B // 128, 128)
              .transpose(0, 2, 1, 3).reshape(-1))
    return fn(x_flat,
              target.astype(jnp.int32),
              weight.astype(jnp.float32))


# speedup vs baseline: 1.0095x; 1.0095x over previous
import functools

import jax
import jax.numpy as jnp
from jax import lax
from jax.experimental import pallas as pl
from jax.experimental.pallas import tpu as pltpu
from jax.experimental.pallas import tpu_sc as plsc

IGNORE_INDEX = -100
LANES = 16
IDX_MINOR = 128


def _sc_workers():
    try:
        info = plsc.get_sparse_core_info()
        return info.num_cores, info.num_subcores
    except Exception:
        return 2, 16


@functools.lru_cache(maxsize=None)
def _build(B: int, C: int):
    NC, NS = _sc_workers()
    NW = NC * NS
    assert B % (NW * IDX_MINOR) == 0 and C % 8 == 0
    assert B & (B - 1) == 0
    SH_T = (8 * B).bit_length() - 1
    b_per_w = B // NW
    n_str = b_per_w // IDX_MINOR
    mesh = plsc.VectorSubcoreMesh(core_axis_name="c", subcore_axis_name="s",
                                  num_cores=NC)

    @functools.partial(
        pl.kernel,
        out_type=jax.ShapeDtypeStruct((B,), jnp.float32),
        mesh=mesh,
        scratch_types=[
            pltpu.VMEM((b_per_w,), jnp.int32),
            pltpu.VMEM((b_per_w,), jnp.int32),
            pltpu.VMEM((b_per_w,), jnp.int32),
            pltpu.VMEM((b_per_w,), jnp.float32),
            pltpu.VMEM((b_per_w,), jnp.float32),
            pltpu.VMEM((b_per_w,), jnp.float32),
            pltpu.SemaphoreType.DMA,
        ],
    )
    def nll_kernel(xf_hbm, tgt_hbm, w_hbm, out_hbm,
                   tgt_v, idx_v, safe_v, picked_v, wpick_v, out_v, sem):
        wid = lax.axis_index("s") * NC + lax.axis_index("c")
        base = wid * b_per_w

        pltpu.sync_copy(tgt_hbm.at[pl.ds(base, b_per_w)], tgt_v)

        lane = lax.iota(jnp.int32, LANES)

        def idx_body(j, _):
            o = j * LANES
            tv = tgt_v[pl.ds(o, LANES)]
            safe_t = jnp.where(tv == IGNORE_INDEX, 0, tv)
            rows = base + o + lane
            safe_v[pl.ds(o, LANES)] = safe_t
            idx_v[pl.ds(o, LANES)] = (
                ((safe_t >> 3) << SH_T)
                + ((rows >> 7) << 10)
                + ((safe_t & 7) << 7)
                + (rows & 127))
            return _

        lax.fori_loop(0, b_per_w // LANES, idx_body, None)

        copies = []
        for r in range(n_str):
            s = pl.ds(r * IDX_MINOR, IDX_MINOR)
            copies.append(
                pltpu.async_copy(xf_hbm.at[idx_v.at[s]], picked_v.at[s], sem))
        for r in range(n_str):
            s = pl.ds(r * IDX_MINOR, IDX_MINOR)
            copies.append(
                pltpu.async_copy(w_hbm.at[safe_v.at[s]], wpick_v.at[s], sem))
        for c in copies:
            c.wait()

        def out_body(j, _):
            o = j * LANES
            tv = tgt_v[pl.ds(o, LANES)]
            valid = tv != IGNORE_INDEX
            wv = wpick_v[pl.ds(o, LANES)]
            pk = picked_v[pl.ds(o, LANES)]
            out_v[pl.ds(o, LANES)] = jnp.where(
                valid, -(wv * pk), jnp.float32(0.0))
            return _

        lax.fori_loop(0, b_per_w // LANES, out_body, None)

        pltpu.sync_copy(out_v, out_hbm.at[pl.ds(base, b_per_w)])

    return nll_kernel


def kernel(x, target, weight):
    B, C = x.shape
    fn = _build(B, C)
    x_flat = (x.T.reshape(C // 8, 8, B // 128, 128)
              .transpose(0, 2, 1, 3).reshape(-1))
    return fn(x_flat,
              target.astype(jnp.int32),
              weight.astype(jnp.float32))


# speedup vs baseline: 1.4054x; 1.3922x over previous
import functools

import jax
import jax.numpy as jnp
from jax import lax
from jax.experimental import pallas as pl
from jax.experimental.pallas import tpu as pltpu
from jax.experimental.pallas import tpu_sc as plsc

IGNORE_INDEX = -100
LANES = 16
IDX_MINOR = 128


def _sc_workers():
    try:
        info = plsc.get_sparse_core_info()
        return info.num_cores, info.num_subcores
    except Exception:
        return 2, 16


@functools.lru_cache(maxsize=None)
def _build(B: int, C: int):
    NC, NS = _sc_workers()
    NW = NC * NS
    assert B % (NW * IDX_MINOR) == 0 and C % 8 == 0
    assert B & (B - 1) == 0
    SH_T = (8 * B).bit_length() - 1
    CP = 1 << (C - 1).bit_length()
    SH_W = CP.bit_length() - 1
    b_per_w = B // NW
    n_str = b_per_w // IDX_MINOR
    mesh = plsc.VectorSubcoreMesh(core_axis_name="c", subcore_axis_name="s",
                                  num_cores=NC)

    @functools.partial(
        pl.kernel,
        out_type=jax.ShapeDtypeStruct((B,), jnp.float32),
        mesh=mesh,
        scratch_types=[
            pltpu.VMEM((b_per_w,), jnp.int32),
            pltpu.VMEM((b_per_w,), jnp.int32),
            pltpu.VMEM((b_per_w,), jnp.int32),
            pltpu.VMEM((b_per_w,), jnp.float32),
            pltpu.VMEM((b_per_w,), jnp.float32),
            pltpu.VMEM((b_per_w,), jnp.float32),
            pltpu.SemaphoreType.DMA,
        ],
    )
    def nll_kernel(xf_hbm, tgt_hbm, w_hbm, out_hbm,
                   tgt_v, idx_v, safe_v, picked_v, wpick_v, out_v, sem):
        wid = lax.axis_index("s") * NC + lax.axis_index("c")
        base = wid * b_per_w

        pltpu.sync_copy(tgt_hbm.at[pl.ds(base, b_per_w)], tgt_v)

        lane = lax.iota(jnp.int32, LANES)

        wbase = wid << SH_W

        def idx_body(j, _):
            o = j * LANES
            tv = tgt_v[pl.ds(o, LANES)]
            safe_t = jnp.where(tv == IGNORE_INDEX, 0, tv)
            rows = base + o + lane
            safe_v[pl.ds(o, LANES)] = wbase + safe_t
            idx_v[pl.ds(o, LANES)] = (
                ((safe_t >> 3) << SH_T)
                + ((rows >> 7) << 10)
                + ((safe_t & 7) << 7)
                + (rows & 127))
            return _

        lax.fori_loop(0, b_per_w // LANES, idx_body, None)

        copies = []
        for r in range(n_str):
            s = pl.ds(r * IDX_MINOR, IDX_MINOR)
            copies.append(
                pltpu.async_copy(xf_hbm.at[idx_v.at[s]], picked_v.at[s], sem))
        for r in range(n_str):
            s = pl.ds(r * IDX_MINOR, IDX_MINOR)
            copies.append(
                pltpu.async_copy(w_hbm.at[safe_v.at[s]], wpick_v.at[s], sem))
        for c in copies:
            c.wait()

        def out_body(j, _):
            o = j * LANES
            tv = tgt_v[pl.ds(o, LANES)]
            valid = tv != IGNORE_INDEX
            wv = wpick_v[pl.ds(o, LANES)]
            pk = picked_v[pl.ds(o, LANES)]
            out_v[pl.ds(o, LANES)] = jnp.where(
                valid, -(wv * pk), jnp.float32(0.0))
            return _

        lax.fori_loop(0, b_per_w // LANES, out_body, None)

        pltpu.sync_copy(out_v, out_hbm.at[pl.ds(base, b_per_w)])

    return nll_kernel


def kernel(x, target, weight):
    B, C = x.shape
    fn = _build(B, C)
    x_flat = (x.T.reshape(C // 8, 8, B // 128, 128)
              .transpose(0, 2, 1, 3).reshape(-1))
    nw, cp = _sc_workers()[0] * _sc_workers()[1], 1 << (C - 1).bit_length()
    w_rep = jnp.broadcast_to(
        jnp.pad(weight.astype(jnp.float32), (0, cp - C)), (nw, cp)).reshape(-1)
    return fn(x_flat,
              target.astype(jnp.int32),
              w_rep)
